# baseline (speedup 1.0000x reference)
import jax
import jax.numpy as jnp
from jax.experimental import pallas as pl
from jax.experimental.pallas import tpu as pltpu

E = 2048
B = 64
BJ = 16
NBITS = 32


def _bits_kernel(xt_ref, out_ref):
    v = 2 * xt_ref[:, :] + 1
    out_ref[:, : E - NBITS, :] = jnp.zeros((BJ, E - NBITS, B), jnp.float32)
    shifts = (NBITS - 1) - jax.lax.broadcasted_iota(jnp.int32, (BJ, NBITS, B), 1)
    bits = (v[:, None, :] >> shifts) & 1
    out_ref[:, E - NBITS :, :] = bits.astype(jnp.float32)


def kernel(x):
    xt = x.T
    return pl.pallas_call(
        _bits_kernel,
        grid=(E // BJ,),
        in_specs=[pl.BlockSpec((BJ, B), lambda j: (j, 0))],
        out_specs=pl.BlockSpec((BJ, E, B), lambda j: (j, 0, 0)),
        out_shape=jax.ShapeDtypeStruct((E, E, B), jnp.float32),
        compiler_params=pltpu.CompilerParams(dimension_semantics=("parallel",)),
    )(xt)


# speedup vs baseline: 1.0468x; 1.0468x over previous
import jax
import jax.numpy as jnp
from jax.experimental import pallas as pl
from jax.experimental.pallas import tpu as pltpu

E = 2048
B = 64
BJ = 16
NBITS = 32
ZCOLS = (E - NBITS) * B


def _bits_kernel(xt_ref, out_ref):
    v = 2 * xt_ref[:, :] + 1
    out_ref[:, :ZCOLS] = jnp.zeros((BJ, ZCOLS), jnp.float32)
    shifts = (NBITS - 1) - jax.lax.broadcasted_iota(
        jnp.int32, (BJ, NBITS, B), 1
    )
    bits = ((v[:, None, :] >> shifts) & 1).astype(jnp.float32)
    out_ref[:, ZCOLS:] = bits.reshape(BJ, NBITS * B)


def kernel(x):
    xt = x.T
    flat = pl.pallas_call(
        _bits_kernel,
        grid=(E // BJ,),
        in_specs=[pl.BlockSpec((BJ, B), lambda j: (j, 0))],
        out_specs=pl.BlockSpec((BJ, E * B), lambda j: (j, 0)),
        out_shape=jax.ShapeDtypeStruct((E, E * B), jnp.float32),
        compiler_params=pltpu.CompilerParams(dimension_semantics=("parallel",)),
    )(xt)
    return flat.reshape(E, E, B)
